# baseline (speedup 1.0000x reference)
import jax
import jax.numpy as jnp
from jax import lax
from jax.experimental import pallas as pl
from jax.experimental.pallas import tpu as pltpu
from jax.experimental.pallas import tpu_sc as plsc

EMBED_DIM = 128
BATCH = 16384
VOCAB_ROWS = 1489
NUM_CORES = 2
NUM_SUBCORES = 16
NUM_WORKERS = NUM_CORES * NUM_SUBCORES
BPW = BATCH // NUM_WORKERS
CHUNK = 128
NCHUNK = BPW // CHUNK
TOTAL = 2 * NCHUNK
TROWS = 96


def _gather_body(table_hbm, ov_hbm, sc_hbm, out_ov, out_sc,
                 table_sh, idx_all, rows_a, rows_b,
                 isem, tsem, gsem_a, gsem_b, wsem_a, wsem_b):
    sid = lax.axis_index("s")
    wid = sid * NUM_CORES + lax.axis_index("c")
    row0 = wid * NCHUNK
    base = wid * BPW

    tail = sid == NUM_SUBCORES - 1

    @pl.when(jnp.logical_not(tail))
    def _stage_main():
        pltpu.async_copy(
            table_hbm.at[pl.ds(sid * TROWS, TROWS)],
            table_sh.at[pl.ds(sid * TROWS, TROWS)], tsem).wait()

    @pl.when(tail)
    def _stage_tail():
        pltpu.async_copy(
            table_hbm.at[pl.ds(15 * TROWS, VOCAB_ROWS - 15 * TROWS)],
            table_sh.at[pl.ds(15 * TROWS, VOCAB_ROWS - 15 * TROWS)],
            tsem).wait()

    cp_i0 = pltpu.async_copy(
        ov_hbm.at[pl.ds(row0, NCHUNK)], idx_all.at[pl.ds(0, NCHUNK)], isem)
    cp_i1 = pltpu.async_copy(
        sc_hbm.at[pl.ds(row0, NCHUNK)], idx_all.at[pl.ds(NCHUNK, NCHUNK)],
        isem)
    cp_i0.wait()
    cp_i1.wait()
    plsc.subcore_barrier()

    def step(i, carry):
        off = base + i * CHUNK

        @pl.when(i > 0)
        def _drain_prev():
            pltpu.make_async_copy(
                out_ov.at[pl.ds(base, CHUNK)], rows_a, wsem_a).wait()
            pltpu.make_async_copy(
                out_sc.at[pl.ds(base, CHUNK)], rows_b, wsem_b).wait()

        ga = pltpu.async_copy(table_sh.at[idx_all.at[i]], rows_a, gsem_a)
        gb = pltpu.async_copy(
            table_sh.at[idx_all.at[i + NCHUNK]], rows_b, gsem_b)
        ga.wait()
        pltpu.async_copy(rows_a, out_ov.at[pl.ds(off, CHUNK)], wsem_a)
        gb.wait()
        pltpu.async_copy(rows_b, out_sc.at[pl.ds(off, CHUNK)], wsem_b)
        return carry

    lax.fori_loop(0, NCHUNK, step, 0)
    pltpu.make_async_copy(
        out_ov.at[pl.ds(base, CHUNK)], rows_a, wsem_a).wait()
    pltpu.make_async_copy(
        out_sc.at[pl.ds(base, CHUNK)], rows_b, wsem_b).wait()


@jax.jit
def kernel(overlap, scene, embed_table):
    ov = overlap.astype(jnp.int32).reshape(BATCH // CHUNK, CHUNK)
    sc = scene.astype(jnp.int32).reshape(BATCH // CHUNK, CHUNK)
    out_sds = jax.ShapeDtypeStruct((BATCH, EMBED_DIM), jnp.float32)
    run = pl.kernel(
        _gather_body,
        out_type=(out_sds, out_sds),
        mesh=plsc.VectorSubcoreMesh(core_axis_name="c", subcore_axis_name="s"),
        scratch_types=(
            [pltpu.VMEM_SHARED((VOCAB_ROWS, EMBED_DIM), jnp.float32)]
            + [pltpu.VMEM((TOTAL, CHUNK), jnp.int32)]
            + [pltpu.VMEM((CHUNK, EMBED_DIM), jnp.float32)] * 2
            + [pltpu.SemaphoreType.DMA] * 6
        ),
    )
    return run(embed_table, ov, sc)
